# baseline (speedup 1.0000x reference)
import functools

import jax
import jax.numpy as jnp
from jax import lax
from jax.experimental import pallas as pl
from jax.experimental.pallas import tpu as pltpu
from jax.experimental.pallas import tpu_sc as plsc

_NC = 2
_NS = 16
_NW = _NC * _NS
_L = 16
_N = 16384
_D = 1024
_F = 3072
_RPW = (_N - _F) // _NW
_G = 16
_DUMP = _RPW + _L
_TC_ROWS = 512


def _sc_body(x_hbm, p_hbm, m_hbm, sums_hbm,
             mask_v, idx_v, xb0, pb0, xb1, pb1, xb2, pb2, accb,
             sem0, sem1, sem2):
    wid = lax.axis_index("s") * _NC + lax.axis_index("c")
    base = _F + wid * _RPW

    pltpu.sync_copy(m_hbm.at[pl.ds(base, _RPW)], mask_v)

    lane = lax.iota(jnp.int32, _L)

    def _compact(i, cnt):
        m = mask_v[pl.ds(i * _L, _L)]
        s = m
        for k in (1, 2, 4, 8):
            sh = s.at[jnp.maximum(lane - k, 0)].get(mode="promise_in_bounds")
            s = s + jnp.where(lane >= k, sh, 0)
        pos = jnp.where(m > 0, cnt + s - m, _DUMP + lane)
        plsc.store_scatter(idx_v, [pos], (base + i * _L) + lane)
        return cnt + s[_L - 1]

    cnt = lax.fori_loop(0, _RPW // _L, _compact, jnp.int32(0))

    idx_v[pl.ds(cnt, _L)] = jnp.full((_L,), base, jnp.int32)

    accb[...] = jnp.zeros((_L,), jnp.float32)
    nchunks = (cnt + (_G - 1)) // _G

    def start(ci, xb, pb, sem):
        isl = idx_v.at[pl.ds(ci * _G, _G)]
        pltpu.async_copy(x_hbm.at[isl], xb, sem)
        pltpu.async_copy(p_hbm.at[isl], pb, sem)

    def waitfor(xb, pb, sem):
        pltpu.make_async_copy(x_hbm.at[pl.ds(0, _G)], xb, sem).wait()
        pltpu.make_async_copy(p_hbm.at[pl.ds(0, _G)], pb, sem).wait()

    def compute(ci, xb, pb):
        vrows = jnp.minimum(cnt - ci * _G, _G)

        def row(j, accs):
            def kb(k, ac):
                a0, a1 = ac
                for u in range(8):
                    off = (k * 8 + u) * _L
                    d = pb[j, pl.ds(off, _L)] - xb[j, pl.ds(off, _L)]
                    if u % 2 == 0:
                        a0 = a0 + d * d
                    else:
                        a1 = a1 + d * d
                return (a0, a1)

            return lax.fori_loop(0, _D // _L // 8, kb, accs)

        z = jnp.zeros((_L,), jnp.float32)
        a0, a1 = lax.fori_loop(0, vrows, row, (z, z))
        accb[...] = accb[...] + a0 + a1

    slots = ((xb0, pb0, sem0), (xb1, pb1, sem1), (xb2, pb2, sem2))
    nbuf = len(slots)

    for b in range(nbuf):
        @pl.when(b < nchunks)
        def _prime(b=b):
            start(b, *slots[b])

    def chunk_group(h, carry):
        c0 = h * nbuf
        for b in range(nbuf):
            xb, pb, sem = slots[b]

            @pl.when(c0 + b < nchunks)
            def _go(b=b, xb=xb, pb=pb, sem=sem):
                waitfor(xb, pb, sem)
                compute(c0 + b, xb, pb)

                @pl.when(c0 + b + nbuf < nchunks)
                def _next():
                    start(c0 + b + nbuf, xb, pb, sem)

        return carry

    lax.fori_loop(0, (nchunks + nbuf - 1) // nbuf, chunk_group, jnp.int32(0))

    pltpu.sync_copy(accb, sums_hbm.at[wid])


def _tc_block(x_ref, p_ref, m_ref, msc_ref, sum_ref, cnt_ref):
    i = pl.program_id(0)

    @pl.when(i == 0)
    def _init():
        sum_ref[0, 0] = 0.0
        cnt_ref[0, 0] = 0.0

    m = m_ref[0]
    d = p_ref[...] - x_ref[...]
    masked = jax.lax.dot_general(
        m, d * d, (((1,), (0,)), ((), ())),
        preferred_element_type=jnp.float32,
    )
    sum_ref[0, 0] += jnp.sum(masked)

    @pl.when(i == 0)
    def _count_all():
        cnt_ref[0, 0] += jnp.sum(msc_ref[...])


@jax.jit
def _masked_mse(xf, pf, mask):
    mi = mask.astype(jnp.int32)
    mf3 = mask.astype(jnp.float32).reshape(_N // _TC_ROWS, 1, _TC_ROWS)

    mesh = plsc.VectorSubcoreMesh(core_axis_name="c", subcore_axis_name="s")
    sc_sums = pl.kernel(
        _sc_body,
        out_type=jax.ShapeDtypeStruct((_NW, _L), jnp.float32),
        mesh=mesh,
        compiler_params=pltpu.CompilerParams(
            needs_layout_passes=False, skip_device_barrier=True
        ),
        scratch_types=[
            pltpu.VMEM((_RPW,), jnp.int32),
            pltpu.VMEM((_RPW + 2 * _L,), jnp.int32),
            pltpu.VMEM((_G, _D), jnp.float32),
            pltpu.VMEM((_G, _D), jnp.float32),
            pltpu.VMEM((_G, _D), jnp.float32),
            pltpu.VMEM((_G, _D), jnp.float32),
            pltpu.VMEM((_G, _D), jnp.float32),
            pltpu.VMEM((_G, _D), jnp.float32),
            pltpu.VMEM((_L,), jnp.float32),
            pltpu.SemaphoreType.DMA,
            pltpu.SemaphoreType.DMA,
            pltpu.SemaphoreType.DMA,
        ],
    )(xf, pf, mi)

    tc_sum, tc_cnt = pl.pallas_call(
        _tc_block,
        grid=(_F // _TC_ROWS,),
        in_specs=[
            pl.BlockSpec((_TC_ROWS, _D), lambda i: (i, 0)),
            pl.BlockSpec((_TC_ROWS, _D), lambda i: (i, 0)),
            pl.BlockSpec((1, 1, _TC_ROWS), lambda i: (i, 0, 0)),
            pl.BlockSpec(
                (_N // _TC_ROWS, 1, _TC_ROWS), lambda i: (0, 0, 0)
            ),
        ],
        out_specs=[
            pl.BlockSpec(memory_space=pltpu.SMEM),
            pl.BlockSpec(memory_space=pltpu.SMEM),
        ],
        out_shape=[
            jax.ShapeDtypeStruct((1, 1), jnp.float32),
            jax.ShapeDtypeStruct((1, 1), jnp.float32),
        ],
    )(xf, pf, mf3, mf3)

    total = tc_sum[0, 0] + jnp.sum(sc_sums)
    cnt = tc_cnt[0, 0] * _D
    loss = total / jnp.maximum(cnt, 1.0)
    return jnp.where(cnt == 0, jnp.asarray(0.0, dtype=xf.dtype), loss)


def kernel(x, predictions, mask):
    b, s, d = x.shape
    n = b * s
    xf = x.reshape(n, d)
    pf = predictions.reshape(n, d)
    return _masked_mse(xf, pf, mask.reshape(n))


# speedup vs baseline: 1.0161x; 1.0161x over previous
import functools

import jax
import jax.numpy as jnp
from jax import lax
from jax.experimental import pallas as pl
from jax.experimental.pallas import tpu as pltpu
from jax.experimental.pallas import tpu_sc as plsc

_NC = 2
_NS = 16
_NW = _NC * _NS
_L = 16
_N = 16384
_D = 1024
_F = 4096
_RPW = (_N - _F) // _NW
_G = 16
_DUMP = _RPW + _L
_TC_ROWS = 512


def _sc_body(x_hbm, p_hbm, m_hbm, sums_hbm,
             mask_v, idx_v, xb0, pb0, xb1, pb1, xb2, pb2, accb,
             sem0, sem1, sem2):
    wid = lax.axis_index("s") * _NC + lax.axis_index("c")
    base = _F + wid * _RPW

    pltpu.sync_copy(m_hbm.at[pl.ds(base, _RPW)], mask_v)

    lane = lax.iota(jnp.int32, _L)

    def _compact(i, cnt):
        m = mask_v[pl.ds(i * _L, _L)]
        s = m
        for k in (1, 2, 4, 8):
            sh = s.at[jnp.maximum(lane - k, 0)].get(mode="promise_in_bounds")
            s = s + jnp.where(lane >= k, sh, 0)
        pos = jnp.where(m > 0, cnt + s - m, _DUMP + lane)
        plsc.store_scatter(idx_v, [pos], (base + i * _L) + lane)
        return cnt + s[_L - 1]

    cnt = lax.fori_loop(0, _RPW // _L, _compact, jnp.int32(0))

    idx_v[pl.ds(cnt, _L)] = jnp.full((_L,), base, jnp.int32)

    accb[...] = jnp.zeros((_L,), jnp.float32)
    nchunks = (cnt + (_G - 1)) // _G

    def start(ci, xb, pb, sem):
        isl = idx_v.at[pl.ds(ci * _G, _G)]
        pltpu.async_copy(x_hbm.at[isl], xb, sem)
        pltpu.async_copy(p_hbm.at[isl], pb, sem)

    def waitfor(xb, pb, sem):
        pltpu.make_async_copy(x_hbm.at[pl.ds(0, _G)], xb, sem).wait()
        pltpu.make_async_copy(p_hbm.at[pl.ds(0, _G)], pb, sem).wait()

    def compute(ci, xb, pb):
        vrows = jnp.minimum(cnt - ci * _G, _G)

        def row(j, accs):
            def kb(k, ac):
                a0, a1 = ac
                for u in range(8):
                    off = (k * 8 + u) * _L
                    d = pb[j, pl.ds(off, _L)] - xb[j, pl.ds(off, _L)]
                    if u % 2 == 0:
                        a0 = a0 + d * d
                    else:
                        a1 = a1 + d * d
                return (a0, a1)

            return lax.fori_loop(0, _D // _L // 8, kb, accs)

        z = jnp.zeros((_L,), jnp.float32)
        a0, a1 = lax.fori_loop(0, vrows, row, (z, z))
        accb[...] = accb[...] + a0 + a1

    slots = ((xb0, pb0, sem0), (xb1, pb1, sem1), (xb2, pb2, sem2))
    nbuf = len(slots)

    for b in range(nbuf):
        @pl.when(b < nchunks)
        def _prime(b=b):
            start(b, *slots[b])

    def chunk_group(h, carry):
        c0 = h * nbuf
        for b in range(nbuf):
            xb, pb, sem = slots[b]

            @pl.when(c0 + b < nchunks)
            def _go(b=b, xb=xb, pb=pb, sem=sem):
                waitfor(xb, pb, sem)
                compute(c0 + b, xb, pb)

                @pl.when(c0 + b + nbuf < nchunks)
                def _next():
                    start(c0 + b + nbuf, xb, pb, sem)

        return carry

    lax.fori_loop(0, (nchunks + nbuf - 1) // nbuf, chunk_group, jnp.int32(0))

    pltpu.sync_copy(accb, sums_hbm.at[wid])


def _tc_block(x_ref, p_ref, m_ref, msc_ref, sum_ref, cnt_ref):
    i = pl.program_id(0)

    @pl.when(i == 0)
    def _init():
        sum_ref[0, 0] = 0.0
        cnt_ref[0, 0] = 0.0

    m = m_ref[0]
    d = p_ref[...] - x_ref[...]
    masked = jax.lax.dot_general(
        m, d * d, (((1,), (0,)), ((), ())),
        preferred_element_type=jnp.float32,
    )
    sum_ref[0, 0] += jnp.sum(masked)

    @pl.when(i == 0)
    def _count_all():
        cnt_ref[0, 0] += jnp.sum(msc_ref[...])


@jax.jit
def _masked_mse(xf, pf, mask):
    mi = mask.astype(jnp.int32)
    mf3 = mask.astype(jnp.float32).reshape(_N // _TC_ROWS, 1, _TC_ROWS)

    mesh = plsc.VectorSubcoreMesh(core_axis_name="c", subcore_axis_name="s")
    sc_sums = pl.kernel(
        _sc_body,
        out_type=jax.ShapeDtypeStruct((_NW, _L), jnp.float32),
        mesh=mesh,
        compiler_params=pltpu.CompilerParams(
            needs_layout_passes=False, skip_device_barrier=True
        ),
        scratch_types=[
            pltpu.VMEM((_RPW,), jnp.int32),
            pltpu.VMEM((_RPW + 2 * _L,), jnp.int32),
            pltpu.VMEM((_G, _D), jnp.float32),
            pltpu.VMEM((_G, _D), jnp.float32),
            pltpu.VMEM((_G, _D), jnp.float32),
            pltpu.VMEM((_G, _D), jnp.float32),
            pltpu.VMEM((_G, _D), jnp.float32),
            pltpu.VMEM((_G, _D), jnp.float32),
            pltpu.VMEM((_L,), jnp.float32),
            pltpu.SemaphoreType.DMA,
            pltpu.SemaphoreType.DMA,
            pltpu.SemaphoreType.DMA,
        ],
    )(xf, pf, mi)

    tc_sum, tc_cnt = pl.pallas_call(
        _tc_block,
        grid=(_F // _TC_ROWS,),
        in_specs=[
            pl.BlockSpec((_TC_ROWS, _D), lambda i: (i, 0)),
            pl.BlockSpec((_TC_ROWS, _D), lambda i: (i, 0)),
            pl.BlockSpec((1, 1, _TC_ROWS), lambda i: (i, 0, 0)),
            pl.BlockSpec(
                (_N // _TC_ROWS, 1, _TC_ROWS), lambda i: (0, 0, 0)
            ),
        ],
        out_specs=[
            pl.BlockSpec(memory_space=pltpu.SMEM),
            pl.BlockSpec(memory_space=pltpu.SMEM),
        ],
        out_shape=[
            jax.ShapeDtypeStruct((1, 1), jnp.float32),
            jax.ShapeDtypeStruct((1, 1), jnp.float32),
        ],
    )(xf, pf, mf3, mf3)

    total = tc_sum[0, 0] + jnp.sum(sc_sums)
    cnt = tc_cnt[0, 0] * _D
    loss = total / jnp.maximum(cnt, 1.0)
    return jnp.where(cnt == 0, jnp.asarray(0.0, dtype=xf.dtype), loss)


def kernel(x, predictions, mask):
    b, s, d = x.shape
    n = b * s
    xf = x.reshape(n, d)
    pf = predictions.reshape(n, d)
    return _masked_mse(xf, pf, mask.reshape(n))


# speedup vs baseline: 1.0254x; 1.0092x over previous
import functools

import jax
import jax.numpy as jnp
from jax import lax
from jax.experimental import pallas as pl
from jax.experimental.pallas import tpu as pltpu
from jax.experimental.pallas import tpu_sc as plsc

_NC = 2
_NS = 16
_NW = _NC * _NS
_L = 16
_N = 16384
_D = 1024
_F = 4608
_RPW = (_N - _F) // _NW
_G = 16
_DUMP = _RPW + _L
_TC_ROWS = 512


def _sc_body(x_hbm, p_hbm, m_hbm, sums_hbm,
             mask_v, idx_v, xb0, pb0, xb1, pb1, xb2, pb2, accb,
             sem0, sem1, sem2):
    wid = lax.axis_index("s") * _NC + lax.axis_index("c")
    base = _F + wid * _RPW

    pltpu.sync_copy(m_hbm.at[pl.ds(base, _RPW)], mask_v)

    lane = lax.iota(jnp.int32, _L)

    def _compact(i, cnt):
        m = mask_v[pl.ds(i * _L, _L)]
        s = m
        for k in (1, 2, 4, 8):
            sh = s.at[jnp.maximum(lane - k, 0)].get(mode="promise_in_bounds")
            s = s + jnp.where(lane >= k, sh, 0)
        pos = jnp.where(m > 0, cnt + s - m, _DUMP + lane)
        plsc.store_scatter(idx_v, [pos], (base + i * _L) + lane)
        return cnt + s[_L - 1]

    cnt = lax.fori_loop(0, _RPW // _L, _compact, jnp.int32(0))

    idx_v[pl.ds(cnt, _L)] = jnp.full((_L,), base, jnp.int32)

    accb[...] = jnp.zeros((_L,), jnp.float32)
    nchunks = (cnt + (_G - 1)) // _G

    def start(ci, xb, pb, sem):
        isl = idx_v.at[pl.ds(ci * _G, _G)]
        pltpu.async_copy(x_hbm.at[isl], xb, sem)
        pltpu.async_copy(p_hbm.at[isl], pb, sem)

    def waitfor(xb, pb, sem):
        pltpu.make_async_copy(x_hbm.at[pl.ds(0, _G)], xb, sem).wait()
        pltpu.make_async_copy(p_hbm.at[pl.ds(0, _G)], pb, sem).wait()

    def compute(ci, xb, pb):
        vrows = jnp.minimum(cnt - ci * _G, _G)

        def row(j, accs):
            def kb(k, ac):
                a0, a1 = ac
                for u in range(8):
                    off = (k * 8 + u) * _L
                    d = pb[j, pl.ds(off, _L)] - xb[j, pl.ds(off, _L)]
                    if u % 2 == 0:
                        a0 = a0 + d * d
                    else:
                        a1 = a1 + d * d
                return (a0, a1)

            return lax.fori_loop(0, _D // _L // 8, kb, accs)

        z = jnp.zeros((_L,), jnp.float32)
        a0, a1 = lax.fori_loop(0, vrows, row, (z, z))
        accb[...] = accb[...] + a0 + a1

    slots = ((xb0, pb0, sem0), (xb1, pb1, sem1), (xb2, pb2, sem2))
    nbuf = len(slots)

    for b in range(nbuf):
        @pl.when(b < nchunks)
        def _prime(b=b):
            start(b, *slots[b])

    def chunk_group(h, carry):
        c0 = h * nbuf
        for b in range(nbuf):
            xb, pb, sem = slots[b]

            @pl.when(c0 + b < nchunks)
            def _go(b=b, xb=xb, pb=pb, sem=sem):
                waitfor(xb, pb, sem)
                compute(c0 + b, xb, pb)

                @pl.when(c0 + b + nbuf < nchunks)
                def _next():
                    start(c0 + b + nbuf, xb, pb, sem)

        return carry

    lax.fori_loop(0, (nchunks + nbuf - 1) // nbuf, chunk_group, jnp.int32(0))

    pltpu.sync_copy(accb, sums_hbm.at[wid])


def _tc_block(x_ref, p_ref, m_ref, msc_ref, sum_ref, cnt_ref):
    i = pl.program_id(0)

    @pl.when(i == 0)
    def _init():
        sum_ref[0, 0] = 0.0
        cnt_ref[0, 0] = 0.0

    m = m_ref[0]
    d = p_ref[...] - x_ref[...]
    masked = jax.lax.dot_general(
        m, d * d, (((1,), (0,)), ((), ())),
        preferred_element_type=jnp.float32,
    )
    sum_ref[0, 0] += jnp.sum(masked)

    @pl.when(i == 0)
    def _count_all():
        cnt_ref[0, 0] += jnp.sum(msc_ref[...])


@jax.jit
def _masked_mse(xf, pf, mask):
    mi = mask.astype(jnp.int32)
    mf3 = mask.astype(jnp.float32).reshape(_N // _TC_ROWS, 1, _TC_ROWS)

    mesh = plsc.VectorSubcoreMesh(core_axis_name="c", subcore_axis_name="s")
    sc_sums = pl.kernel(
        _sc_body,
        out_type=jax.ShapeDtypeStruct((_NW, _L), jnp.float32),
        mesh=mesh,
        compiler_params=pltpu.CompilerParams(
            needs_layout_passes=False, skip_device_barrier=True
        ),
        scratch_types=[
            pltpu.VMEM((_RPW,), jnp.int32),
            pltpu.VMEM((_RPW + 2 * _L,), jnp.int32),
            pltpu.VMEM((_G, _D), jnp.float32),
            pltpu.VMEM((_G, _D), jnp.float32),
            pltpu.VMEM((_G, _D), jnp.float32),
            pltpu.VMEM((_G, _D), jnp.float32),
            pltpu.VMEM((_G, _D), jnp.float32),
            pltpu.VMEM((_G, _D), jnp.float32),
            pltpu.VMEM((_L,), jnp.float32),
            pltpu.SemaphoreType.DMA,
            pltpu.SemaphoreType.DMA,
            pltpu.SemaphoreType.DMA,
        ],
    )(xf, pf, mi)

    tc_sum, tc_cnt = pl.pallas_call(
        _tc_block,
        grid=(_F // _TC_ROWS,),
        in_specs=[
            pl.BlockSpec((_TC_ROWS, _D), lambda i: (i, 0)),
            pl.BlockSpec((_TC_ROWS, _D), lambda i: (i, 0)),
            pl.BlockSpec((1, 1, _TC_ROWS), lambda i: (i, 0, 0)),
            pl.BlockSpec(
                (_N // _TC_ROWS, 1, _TC_ROWS), lambda i: (0, 0, 0)
            ),
        ],
        out_specs=[
            pl.BlockSpec(memory_space=pltpu.SMEM),
            pl.BlockSpec(memory_space=pltpu.SMEM),
        ],
        out_shape=[
            jax.ShapeDtypeStruct((1, 1), jnp.float32),
            jax.ShapeDtypeStruct((1, 1), jnp.float32),
        ],
    )(xf, pf, mf3, mf3)

    total = tc_sum[0, 0] + jnp.sum(sc_sums)
    cnt = tc_cnt[0, 0] * _D
    loss = total / jnp.maximum(cnt, 1.0)
    return jnp.where(cnt == 0, jnp.asarray(0.0, dtype=xf.dtype), loss)


def kernel(x, predictions, mask):
    b, s, d = x.shape
    n = b * s
    xf = x.reshape(n, d)
    pf = predictions.reshape(n, d)
    return _masked_mse(xf, pf, mask.reshape(n))
